# baseline (device time: 33884 ns/iter reference)
import jax
import jax.numpy as jnp
from jax import lax
from jax.experimental import pallas as pl
from jax.experimental.pallas import tpu as pltpu

N_DEV = 4
N_LAYERS = 3
WCHUNKS = 4
FROM_LEFT, FROM_RIGHT, FROM_DIAG = 0, 1, 2


def kernel(x, Win0, Wout0, Win1, Wout1, Win2, Wout2):
    b, d = x.shape
    out_rows = b // N_DEV

    def body(x_ref, win0, wout0, win1, wout1, win2, wout2,
             out_ref, partial_ref, comm_ref, win_buf, wout_buf,
             win_bf, wout_bf, send_sems, recv_sems,
             win_dma_sems, wout_dma_sems):
        my = lax.axis_index("i")
        left = lax.rem(my + N_DEV - 1, N_DEV)
        right = lax.rem(my + 1, N_DEV)
        diag = lax.rem(my + 2, N_DEV)

        wins = [win0, win1, win2]
        wouts = [wout0, wout1, wout2]

        def start_weight_dma(l):
            copies = []
            for c in range(WCHUNKS):
                rw = pl.ds(c * (win_buf.shape[1] // WCHUNKS),
                           win_buf.shape[1] // WCHUNKS)
                cw = pltpu.make_async_copy(
                    wins[l].at[rw], win_buf.at[l % 2, rw],
                    win_dma_sems.at[l % 2, c])
                ro = pl.ds(c * (wout_buf.shape[1] // WCHUNKS),
                           wout_buf.shape[1] // WCHUNKS)
                co = pltpu.make_async_copy(
                    wouts[l].at[ro], wout_buf.at[l % 2, ro],
                    wout_dma_sems.at[l % 2, c])
                cw.start()
                co.start()
                copies.append((cw, co))
            return copies

        def wait_and_convert(l, copies):
            for cw, co in copies:
                cw.wait()
                co.wait()
            win_bf[l % 2] = win_buf[l % 2].astype(jnp.bfloat16)
            wout_bf[l % 2] = wout_buf[l % 2].astype(jnp.bfloat16)

        pending = start_weight_dma(0)

        barrier = pltpu.get_barrier_semaphore()
        for nbr in (left, right, diag):
            pl.semaphore_signal(
                barrier, inc=1,
                device_id=(nbr,), device_id_type=pl.DeviceIdType.MESH,
            )
        pl.semaphore_wait(barrier, 3)

        wait_and_convert(0, pending)
        xb = x_ref[...].astype(jnp.bfloat16)
        for l in range(N_LAYERS):
            if l + 1 < N_LAYERS:
                pending = start_weight_dma(l + 1)

            h = jnp.dot(xb, win_bf[l % 2],
                        preferred_element_type=jnp.float32)
            h = jnp.maximum(h, 0.0).astype(jnp.bfloat16)
            part = jnp.dot(h, wout_bf[l % 2],
                           preferred_element_type=jnp.float32)
            partial_ref[...] = part.astype(jnp.bfloat16)

            rdmas = []
            for j, (peer, slot) in enumerate((
                (left, FROM_RIGHT), (right, FROM_LEFT), (diag, FROM_DIAG),
            )):
                r = pltpu.make_async_remote_copy(
                    src_ref=partial_ref,
                    dst_ref=comm_ref.at[l, slot],
                    send_sem=send_sems.at[l, j],
                    recv_sem=recv_sems.at[l, slot],
                    device_id=(peer,), device_id_type=pl.DeviceIdType.MESH,
                )
                r.start()
                rdmas.append(r)

            if l + 1 < N_LAYERS:
                wait_and_convert(l + 1, pending)

            for r in rdmas:
                r.wait_recv()

            if l < N_LAYERS - 1:
                total = (part
                         + comm_ref[l, FROM_LEFT].astype(jnp.float32)
                         + comm_ref[l, FROM_RIGHT].astype(jnp.float32)
                         + comm_ref[l, FROM_DIAG].astype(jnp.float32))
                xb = total.astype(jnp.bfloat16)
            else:
                rows = pl.ds(my * out_rows, out_rows)
                out_ref[...] = (
                    partial_ref[rows, :].astype(jnp.float32)
                    + comm_ref[l, FROM_LEFT, rows, :].astype(jnp.float32)
                    + comm_ref[l, FROM_RIGHT, rows, :].astype(jnp.float32)
                    + comm_ref[l, FROM_DIAG, rows, :].astype(jnp.float32))

            for r in rdmas:
                r.wait_send()

    d_in, h_in = Win0.shape
    return pl.pallas_call(
        body,
        out_shape=jax.ShapeDtypeStruct((out_rows, d), jnp.float32),
        in_specs=[pl.BlockSpec(memory_space=pltpu.VMEM)]
        + [pl.BlockSpec(memory_space=pl.ANY)] * 6,
        out_specs=pl.BlockSpec(memory_space=pltpu.VMEM),
        scratch_shapes=[
            pltpu.VMEM((b, d), jnp.bfloat16),
            pltpu.VMEM((N_LAYERS, 3, b, d), jnp.bfloat16),
            pltpu.VMEM((2, d_in, h_in), jnp.float32),
            pltpu.VMEM((2, h_in, d_in), jnp.float32),
            pltpu.VMEM((2, d_in, h_in), jnp.bfloat16),
            pltpu.VMEM((2, h_in, d_in), jnp.bfloat16),
            pltpu.SemaphoreType.DMA((N_LAYERS, 3)),
            pltpu.SemaphoreType.DMA((N_LAYERS, 3)),
            pltpu.SemaphoreType.DMA((2, WCHUNKS)),
            pltpu.SemaphoreType.DMA((2, WCHUNKS)),
        ],
        compiler_params=pltpu.CompilerParams(
            collective_id=0,
            vmem_limit_bytes=100 * 1024 * 1024,
        ),
    )(x, Win0, Wout0, Win1, Wout1, Win2, Wout2)


# device time: 29039 ns/iter; 1.1668x vs baseline; 1.1668x over previous
import jax
import jax.numpy as jnp
from jax import lax
from jax.experimental import pallas as pl
from jax.experimental.pallas import tpu as pltpu

N_DEV = 4
N_LAYERS = 3
WCHUNKS = 4
CCHUNKS = 2
FROM_LEFT, FROM_RIGHT, FROM_DIAG = 0, 1, 2


def kernel(x, Win0, Wout0, Win1, Wout1, Win2, Wout2):
    b, d = x.shape
    out_rows = b // N_DEV
    dc = d // CCHUNKS

    def body(x_ref, win0, wout0, win1, wout1, win2, wout2,
             out_ref, partial_ref, comm_ref, comm3_ref, win_buf, wout_buf,
             send_sems, recv_sems, win_dma_sems, wout_dma_sems):
        my = lax.axis_index("i")
        left = lax.rem(my + N_DEV - 1, N_DEV)
        right = lax.rem(my + 1, N_DEV)
        diag = lax.rem(my + 2, N_DEV)
        peers = ((left, FROM_RIGHT), (right, FROM_LEFT), (diag, FROM_DIAG))

        wins = [win0, win1, win2]
        wouts = [wout0, wout1, wout2]

        def start_weight_dma(l):
            copies = []
            for c in range(WCHUNKS):
                rw = pl.ds(c * (win_buf.shape[1] // WCHUNKS),
                           win_buf.shape[1] // WCHUNKS)
                cw = pltpu.make_async_copy(
                    wins[l].at[rw], win_buf.at[l % 2, rw],
                    win_dma_sems.at[l % 2, c])
                ro = pl.ds(c * (wout_buf.shape[1] // WCHUNKS),
                           wout_buf.shape[1] // WCHUNKS)
                co = pltpu.make_async_copy(
                    wouts[l].at[ro], wout_buf.at[l % 2, ro],
                    wout_dma_sems.at[l % 2, c])
                cw.start()
                co.start()
                copies.append((cw, co))
            return copies

        pending = start_weight_dma(0)

        barrier = pltpu.get_barrier_semaphore()
        for nbr, _ in peers:
            pl.semaphore_signal(
                barrier, inc=1,
                device_id=(nbr,), device_id_type=pl.DeviceIdType.MESH,
            )
        pl.semaphore_wait(barrier, 3)

        xb = x_ref[...].astype(jnp.bfloat16)
        for l in range(N_LAYERS):
            copies = pending
            if l + 1 < N_LAYERS:
                pending = start_weight_dma(l + 1)

            for cw, _ in copies:
                cw.wait()
            h = jnp.dot(xb, win_buf[l % 2].astype(jnp.bfloat16),
                        preferred_element_type=jnp.float32)
            h = jnp.maximum(h, 0.0).astype(jnp.bfloat16)
            for _, co in copies:
                co.wait()

            rdmas = []
            part_cols = []
            if l < N_LAYERS - 1:
                for c in range(CCHUNKS):
                    cols = pl.ds(c * dc, dc)
                    pc = jnp.dot(
                        h, wout_buf[l % 2, :, c * dc:(c + 1) * dc]
                        .astype(jnp.bfloat16),
                        preferred_element_type=jnp.float32)
                    part_cols.append(pc)
                    partial_ref[:, cols] = pc.astype(jnp.bfloat16)
                    for j, (peer, slot) in enumerate(peers):
                        r = pltpu.make_async_remote_copy(
                            src_ref=partial_ref.at[:, cols],
                            dst_ref=comm_ref.at[l, slot, :, cols],
                            send_sem=send_sems.at[l, c, j],
                            recv_sem=recv_sems.at[l, c, slot],
                            device_id=(peer,),
                            device_id_type=pl.DeviceIdType.MESH,
                        )
                        r.start()
                        rdmas.append(r)

                for r in rdmas:
                    r.wait_recv()
                part = jnp.concatenate(part_cols, axis=1)
                total = (part
                         + comm_ref[l, FROM_LEFT].astype(jnp.float32)
                         + comm_ref[l, FROM_RIGHT].astype(jnp.float32)
                         + comm_ref[l, FROM_DIAG].astype(jnp.float32))
                xb = total.astype(jnp.bfloat16)
            else:
                part = jnp.dot(h, wout_buf[l % 2].astype(jnp.bfloat16),
                               preferred_element_type=jnp.float32)
                partial_ref[...] = part.astype(jnp.bfloat16)
                for j, (peer, slot) in enumerate(peers):
                    r = pltpu.make_async_remote_copy(
                        src_ref=partial_ref.at[pl.ds(peer * out_rows,
                                                     out_rows)],
                        dst_ref=comm3_ref.at[slot],
                        send_sem=send_sems.at[l, 0, j],
                        recv_sem=recv_sems.at[l, 0, slot],
                        device_id=(peer,),
                        device_id_type=pl.DeviceIdType.MESH,
                    )
                    r.start()
                    rdmas.append(r)
                for r in rdmas:
                    r.wait_recv()
                rows = pl.ds(my * out_rows, out_rows)
                out_ref[...] = (
                    partial_ref[rows, :].astype(jnp.float32)
                    + comm3_ref[FROM_LEFT].astype(jnp.float32)
                    + comm3_ref[FROM_RIGHT].astype(jnp.float32)
                    + comm3_ref[FROM_DIAG].astype(jnp.float32))

            for r in rdmas:
                r.wait_send()

    d_in, h_in = Win0.shape
    return pl.pallas_call(
        body,
        out_shape=jax.ShapeDtypeStruct((out_rows, d), jnp.float32),
        in_specs=[pl.BlockSpec(memory_space=pltpu.VMEM)]
        + [pl.BlockSpec(memory_space=pl.ANY)] * 6,
        out_specs=pl.BlockSpec(memory_space=pltpu.VMEM),
        scratch_shapes=[
            pltpu.VMEM((b, d), jnp.bfloat16),
            pltpu.VMEM((N_LAYERS - 1, 3, b, d), jnp.bfloat16),
            pltpu.VMEM((3, out_rows, d), jnp.bfloat16),
            pltpu.VMEM((2, d_in, h_in), jnp.float32),
            pltpu.VMEM((2, h_in, d_in), jnp.float32),
            pltpu.SemaphoreType.DMA((N_LAYERS, CCHUNKS, 3)),
            pltpu.SemaphoreType.DMA((N_LAYERS, CCHUNKS, 3)),
            pltpu.SemaphoreType.DMA((2, WCHUNKS)),
            pltpu.SemaphoreType.DMA((2, WCHUNKS)),
        ],
        compiler_params=pltpu.CompilerParams(
            collective_id=0,
            vmem_limit_bytes=100 * 1024 * 1024,
        ),
    )(x, Win0, Wout0, Win1, Wout1, Win2, Wout2)
